# Initial kernel scaffold; baseline (speedup 1.0000x reference)
import jax
import jax.numpy as jnp
from jax.experimental import pallas as pl


def kernel(x, edge_index):
    raise NotImplementedError("write your pallas kernel here")


# speedup vs baseline: 5.2242x; 5.2242x over previous
import functools

import jax
import jax.numpy as jnp
from jax import lax
from jax.experimental import pallas as pl
from jax.experimental.pallas import tpu as pltpu
from jax.experimental.pallas import tpu_sc as plsc

N_NODES = 10000
N_EDGES = 320000
D_FEAT = 128

NC = 2
NS = 16
NW = NC * NS
L = 16

D_AUG = D_FEAT + L
CHUNK = 128
CH_PER_W = 79
E_PAD = NW * CH_PER_W * CHUNK
N_PAD = 10240
ROWS_PER_TILE = N_PAD // NS


def _sc_scatter(x_aug, row_pad, col_pad):
    mesh = plsc.VectorSubcoreMesh(core_axis_name="c", subcore_axis_name="s")

    @functools.partial(
        pl.kernel,
        mesh=mesh,
        compiler_params=pltpu.CompilerParams(use_tc_tiling_on_sc=False),
        out_type=jax.ShapeDtypeStruct((NC * N_PAD, D_AUG), jnp.float32),
        scratch_types=[
            pltpu.VMEM_SHARED((N_PAD, D_AUG), jnp.float32),
            pltpu.VMEM((CHUNK,), jnp.int32),
            pltpu.VMEM((CHUNK,), jnp.int32),
            pltpu.VMEM((CHUNK, D_AUG), jnp.float32),
            pltpu.VMEM((L, D_AUG), jnp.float32),
            pltpu.SemaphoreType.DMA,
        ],
    )
    def k(x_hbm, row_hbm, col_hbm, acc_out,
          acc_sp, row_v, col_v, msgs_v, zblk, sem):
        cid = lax.axis_index("c")
        sid = lax.axis_index("s")
        wid = sid * NC + cid

        zero = jnp.zeros((L,), jnp.float32)
        for r in range(L):
            for cc in range(D_AUG // L):
                zblk[r, pl.ds(cc * L, L)] = zero

        tbase = sid * ROWS_PER_TILE

        def zero_body(j, _):
            pltpu.sync_copy(zblk, acc_sp.at[pl.ds(tbase + j * L, L)])
            return _

        lax.fori_loop(0, ROWS_PER_TILE // L, zero_body, None)
        plsc.subcore_barrier()

        def edge_body(j, _):
            off = (wid * CH_PER_W + j) * CHUNK
            pltpu.sync_copy(row_hbm.at[pl.ds(off, CHUNK)], row_v)
            pltpu.sync_copy(col_hbm.at[pl.ds(off, CHUNK)], col_v)
            pltpu.async_copy(x_hbm.at[row_v], msgs_v, sem).wait()
            pltpu.sync_copy(msgs_v, acc_sp.at[col_v], add=True)
            return _

        lax.fori_loop(0, CH_PER_W, edge_body, None)
        plsc.subcore_barrier()

        obase = cid * N_PAD + tbase
        pltpu.sync_copy(acc_sp.at[pl.ds(tbase, ROWS_PER_TILE)],
                        acc_out.at[pl.ds(obase, ROWS_PER_TILE)])

    return k(x_aug, row_pad, col_pad)


def _combine_body(acc_ref, out_ref):
    s = acc_ref[:N_NODES, :] + acc_ref[N_PAD:N_PAD + N_NODES, :]
    out_ref[...] = s[:, :D_FEAT] / jnp.maximum(s[:, D_FEAT:D_FEAT + 1], 1.0)


def kernel(x, edge_index):
    row = edge_index[0]
    col = edge_index[1]
    pad = E_PAD - N_EDGES
    row_pad = jnp.concatenate([row, jnp.zeros((pad,), jnp.int32)])
    col_pad = jnp.concatenate([col, jnp.full((pad,), N_NODES, jnp.int32)])
    ones_col = jnp.zeros((N_NODES, L), jnp.float32).at[:, 0].set(1.0)
    x_aug = jnp.concatenate([x, ones_col], axis=1)

    acc = _sc_scatter(x_aug, row_pad, col_pad)

    out = pl.pallas_call(
        _combine_body,
        out_shape=jax.ShapeDtypeStruct((N_NODES, D_FEAT), jnp.float32),
    )(acc)
    return out
